# Initial kernel scaffold; baseline (speedup 1.0000x reference)
import functools

import jax
import jax.numpy as jnp
from jax import lax
from jax.experimental import pallas as pl
from jax.experimental.pallas import tpu as pltpu
from jax.experimental.pallas import tpu_sc as plsc

N_NODES = 10000
N_EDGES = 320000
D = 128
SLOPE = (1.0 / 8.0 + 1.0 / 3.0) / 2.0

NC = 2
NS = 16
NW = NC * NS
EDGES_PER_W = N_EDGES // NW
CHUNK = 80
N_CHUNKS = EDGES_PER_W // CHUNK
ROWS_PER_SUB = N_NODES // NS
ZROWS = 125

MM_BLOCK = 1000


def _mm_body(x_ref, w_ref, o_ref):
    o_ref[...] = jnp.dot(x_ref[...], w_ref[...],
                         preferred_element_type=jnp.float32)


def _matmul(x, w):
    return pl.pallas_call(
        _mm_body,
        grid=(N_NODES // MM_BLOCK,),
        in_specs=[pl.BlockSpec((MM_BLOCK, D), lambda i: (i, 0)),
                  pl.BlockSpec((D, D), lambda i: (0, 0))],
        out_specs=pl.BlockSpec((MM_BLOCK, D), lambda i: (i, 0)),
        out_shape=jax.ShapeDtypeStruct((N_NODES, D), jnp.float32),
    )(x, w)


def _add_lrelu_mm_body(a_ref, b_ref, w_ref, o_ref):
    h = a_ref[...] + b_ref[...]
    h = jnp.maximum(h, SLOPE * h)
    o_ref[...] = jnp.dot(h, w_ref[...], preferred_element_type=jnp.float32)


def _add_lrelu_matmul(a, b, w):
    return pl.pallas_call(
        _add_lrelu_mm_body,
        grid=(N_NODES // MM_BLOCK,),
        in_specs=[pl.BlockSpec((MM_BLOCK, D), lambda i: (i, 0)),
                  pl.BlockSpec((MM_BLOCK, D), lambda i: (i, 0)),
                  pl.BlockSpec((D, D), lambda i: (0, 0))],
        out_specs=pl.BlockSpec((MM_BLOCK, D), lambda i: (i, 0)),
        out_shape=jax.ShapeDtypeStruct((N_NODES, D), jnp.float32),
    )(a, b, w)


def _add_lrelu_body(a_ref, b_ref, o_ref):
    h = a_ref[...] + b_ref[...]
    o_ref[...] = jnp.maximum(h, SLOPE * h)


def _add_lrelu(a, b):
    return pl.pallas_call(
        _add_lrelu_body,
        grid=(N_NODES // MM_BLOCK,),
        in_specs=[pl.BlockSpec((MM_BLOCK, D), lambda i: (i, 0)),
                  pl.BlockSpec((MM_BLOCK, D), lambda i: (i, 0))],
        out_specs=pl.BlockSpec((MM_BLOCK, D), lambda i: (i, 0)),
        out_shape=jax.ShapeDtypeStruct((N_NODES, D), jnp.float32),
    )(a, b)


def _seg_sum_sc(h, src, dst):
    mesh = plsc.VectorSubcoreMesh(core_axis_name="c", subcore_axis_name="s")

    @functools.partial(
        pl.kernel,
        mesh=mesh,
        out_type=[jax.ShapeDtypeStruct((N_NODES, D), jnp.float32),
                  jax.ShapeDtypeStruct((N_NODES, D), jnp.float32)],
        scratch_types=[
            pltpu.VMEM((CHUNK,), jnp.int32),
            pltpu.VMEM((CHUNK,), jnp.int32),
            pltpu.VMEM((CHUNK, D), jnp.float32),
            pltpu.VMEM((ZROWS, D), jnp.float32),
            pltpu.VMEM_SHARED((N_NODES, D), jnp.float32),
        ],
    )
    def k(h_hbm, src_hbm, dst_hbm, out0_hbm, out1_hbm,
          src_v, dst_v, rows_v, zbuf, acc):
        c = lax.axis_index("c")
        s = lax.axis_index("s")
        wid = c * NS + s

        @pl.loop(0, ZROWS)
        def _(r):
            @pl.loop(0, D // 16)
            def _(j):
                zbuf[r, pl.ds(j * 16, 16)] = jnp.zeros((16,), jnp.float32)

        @pl.loop(0, ROWS_PER_SUB // ZROWS)
        def _(z):
            pltpu.sync_copy(
                zbuf, acc.at[pl.ds(s * ROWS_PER_SUB + z * ZROWS, ZROWS)])

        plsc.subcore_barrier()

        @pl.loop(0, N_CHUNKS)
        def _(j):
            ebase = wid * EDGES_PER_W + j * CHUNK
            pltpu.sync_copy(src_hbm.at[pl.ds(ebase, CHUNK)], src_v)
            pltpu.sync_copy(dst_hbm.at[pl.ds(ebase, CHUNK)], dst_v)
            pltpu.sync_copy(h_hbm.at[src_v], rows_v)
            pltpu.sync_copy(rows_v, acc.at[dst_v], add=True)

        plsc.subcore_barrier()

        rbase = s * ROWS_PER_SUB

        @pl.when(c == 0)
        def _():
            pltpu.sync_copy(acc.at[pl.ds(rbase, ROWS_PER_SUB)],
                            out0_hbm.at[pl.ds(rbase, ROWS_PER_SUB)])

        @pl.when(c == 1)
        def _():
            pltpu.sync_copy(acc.at[pl.ds(rbase, ROWS_PER_SUB)],
                            out1_hbm.at[pl.ds(rbase, ROWS_PER_SUB)])

    return k(h, src, dst)


def kernel(feat, edge_index, W1, W2):
    edge_index = edge_index.astype(jnp.int32)
    src = edge_index[0]
    dst = edge_index[1]

    h1 = _matmul(feat, W1)
    a0, a1 = _seg_sum_sc(h1, src, dst)
    h2 = _add_lrelu_matmul(a0, a1, W2)
    b0, b1 = _seg_sum_sc(h2, src, dst)
    return _add_lrelu(b0, b1)


# speedup vs baseline: 5.1653x; 5.1653x over previous
import functools

import jax
import jax.numpy as jnp
from jax import lax
from jax.experimental import pallas as pl
from jax.experimental.pallas import tpu as pltpu
from jax.experimental.pallas import tpu_sc as plsc

N_NODES = 10000
N_EDGES = 320000
D = 128
SLOPE = (1.0 / 8.0 + 1.0 / 3.0) / 2.0

NC = 2
NS = 16
NW = NC * NS
EDGES_PER_W = N_EDGES // NW
CHUNK = 80
N_CHUNKS = EDGES_PER_W // CHUNK
ROWS_MAIN = 640
ROWS_LAST = 400
ZROWS = 80

MM_BLOCK = 1000


def _mm_body(x_ref, w_ref, o_ref):
    o_ref[...] = jnp.dot(x_ref[...], w_ref[...],
                         preferred_element_type=jnp.float32)


def _matmul(x, w):
    return pl.pallas_call(
        _mm_body,
        grid=(N_NODES // MM_BLOCK,),
        in_specs=[pl.BlockSpec((MM_BLOCK, D), lambda i: (i, 0)),
                  pl.BlockSpec((D, D), lambda i: (0, 0))],
        out_specs=pl.BlockSpec((MM_BLOCK, D), lambda i: (i, 0)),
        out_shape=jax.ShapeDtypeStruct((N_NODES, D), jnp.float32),
    )(x, w)


def _add_lrelu_mm_body(a_ref, b_ref, w_ref, o_ref):
    h = a_ref[...] + b_ref[...]
    h = jnp.maximum(h, SLOPE * h)
    o_ref[...] = jnp.dot(h, w_ref[...], preferred_element_type=jnp.float32)


def _add_lrelu_matmul(a, b, w):
    return pl.pallas_call(
        _add_lrelu_mm_body,
        grid=(N_NODES // MM_BLOCK,),
        in_specs=[pl.BlockSpec((MM_BLOCK, D), lambda i: (i, 0)),
                  pl.BlockSpec((MM_BLOCK, D), lambda i: (i, 0)),
                  pl.BlockSpec((D, D), lambda i: (0, 0))],
        out_specs=pl.BlockSpec((MM_BLOCK, D), lambda i: (i, 0)),
        out_shape=jax.ShapeDtypeStruct((N_NODES, D), jnp.float32),
    )(a, b, w)


def _add_lrelu_body(a_ref, b_ref, o_ref):
    h = a_ref[...] + b_ref[...]
    o_ref[...] = jnp.maximum(h, SLOPE * h)


def _add_lrelu(a, b):
    return pl.pallas_call(
        _add_lrelu_body,
        grid=(N_NODES // MM_BLOCK,),
        in_specs=[pl.BlockSpec((MM_BLOCK, D), lambda i: (i, 0)),
                  pl.BlockSpec((MM_BLOCK, D), lambda i: (i, 0))],
        out_specs=pl.BlockSpec((MM_BLOCK, D), lambda i: (i, 0)),
        out_shape=jax.ShapeDtypeStruct((N_NODES, D), jnp.float32),
    )(a, b)


def _seg_sum_sc(h, src, dst):
    mesh = plsc.VectorSubcoreMesh(core_axis_name="c", subcore_axis_name="s")

    @functools.partial(
        pl.kernel,
        mesh=mesh,
        out_type=[jax.ShapeDtypeStruct((N_NODES, D), jnp.float32),
                  jax.ShapeDtypeStruct((N_NODES, D), jnp.float32)],
        scratch_types=[
            pltpu.VMEM((CHUNK,), jnp.int32),
            pltpu.VMEM((CHUNK,), jnp.int32),
            pltpu.VMEM((CHUNK, D), jnp.float32),
            pltpu.VMEM((ZROWS, D), jnp.float32),
            pltpu.VMEM_SHARED((N_NODES, D), jnp.float32),
        ],
    )
    def k(h_hbm, src_hbm, dst_hbm, out0_hbm, out1_hbm,
          src_v, dst_v, rows_v, zbuf, acc):
        c = lax.axis_index("c")
        s = lax.axis_index("s")
        wid = c * NS + s

        @pl.loop(0, ZROWS)
        def _(r):
            @pl.loop(0, D // 16)
            def _(j):
                zbuf[r, pl.ds(j * 16, 16)] = jnp.zeros((16,), jnp.float32)

        @pl.loop(0, ROWS_MAIN // ZROWS)
        def _(z):
            @pl.when(jnp.logical_or(s < NS - 1, z < ROWS_LAST // ZROWS))
            def _():
                pltpu.sync_copy(
                    zbuf, acc.at[pl.ds(s * ROWS_MAIN + z * ZROWS, ZROWS)])

        plsc.subcore_barrier()

        @pl.loop(0, N_CHUNKS)
        def _(j):
            ebase = wid * EDGES_PER_W + j * CHUNK
            pltpu.sync_copy(src_hbm.at[pl.ds(ebase, CHUNK)], src_v)
            pltpu.sync_copy(dst_hbm.at[pl.ds(ebase, CHUNK)], dst_v)
            pltpu.sync_copy(h_hbm.at[src_v], rows_v)
            pltpu.sync_copy(rows_v, acc.at[dst_v], add=True)

        plsc.subcore_barrier()

        rbase = s * ROWS_MAIN

        @pl.when(c == 0)
        def _():
            pltpu.sync_copy(acc.at[pl.ds(rbase, ROWS_LAST)],
                            out0_hbm.at[pl.ds(rbase, ROWS_LAST)])

            @pl.when(s < NS - 1)
            def _():
                pltpu.sync_copy(
                    acc.at[pl.ds(rbase + ROWS_LAST, ROWS_MAIN - ROWS_LAST)],
                    out0_hbm.at[pl.ds(rbase + ROWS_LAST, ROWS_MAIN - ROWS_LAST)])

        @pl.when(c == 1)
        def _():
            pltpu.sync_copy(acc.at[pl.ds(rbase, ROWS_LAST)],
                            out1_hbm.at[pl.ds(rbase, ROWS_LAST)])

            @pl.when(s < NS - 1)
            def _():
                pltpu.sync_copy(
                    acc.at[pl.ds(rbase + ROWS_LAST, ROWS_MAIN - ROWS_LAST)],
                    out1_hbm.at[pl.ds(rbase + ROWS_LAST, ROWS_MAIN - ROWS_LAST)])

    return k(h, src, dst)


def kernel(feat, edge_index, W1, W2):
    edge_index = edge_index.astype(jnp.int32)
    src = edge_index[0]
    dst = edge_index[1]

    h1 = _matmul(feat, W1)
    a0, a1 = _seg_sum_sc(h1, src, dst)
    h2 = _add_lrelu_matmul(a0, a1, W2)
    b0, b1 = _seg_sum_sc(h2, src, dst)
    return _add_lrelu(b0, b1)
